# Initial kernel scaffold; baseline (speedup 1.0000x reference)
import jax
import jax.numpy as jnp
from jax.experimental import pallas as pl


def kernel(d_emb, t_emb, t_mask, y, W_d, b_d, dg, db, Wf, bf, fg, fb, Wa, ba, Wp, bp, pg, pb, cb_d, cb_t, We1, be1, eg, eb, We2, be2):
    raise NotImplementedError("write your pallas kernel here")


# speedup vs baseline: 1.2802x; 1.2802x over previous
import functools
import math

import jax
import jax.numpy as jnp
from jax import lax
from jax.experimental import pallas as pl
from jax.experimental.pallas import tpu as pltpu
from jax.experimental.pallas import tpu_sc as plsc

B = 64; L = 256; D_T = 1280; D_D = 2048; H = 768; OUT = 1024
K_CB = 8192; CODE = 64; NC = 5
KT = 9
LP = L + KT - 1
RSQ = 1.0 / math.sqrt(1.0 + 1e-05)
NVEC = (2 * B * OUT) // CODE
CB_CH = 1024
F32_MIN = float(jnp.finfo(jnp.float32).min)


def _leaky(x):
    return jnp.where(x >= 0, x, 0.01 * x)


def _tenc_body(x_ref, wf_ref, wa_ref, fs_ref, fb_ref, ba_ref, m_ref,
               wsum_ref, gmax_ref):
    x = x_ref[0]
    ct = wf_ref.shape[1]
    accf = jnp.zeros((L, ct), jnp.float32)
    acca = jnp.zeros((L, ct), jnp.float32)
    for k in range(KT):
        xs = x[k:k + L, :]
        accf += jnp.dot(xs, wf_ref[k * D_T:(k + 1) * D_T, :],
                        preferred_element_type=jnp.float32)
        acca += jnp.dot(xs, wa_ref[k * D_T:(k + 1) * D_T, :],
                        preferred_element_type=jnp.float32)
    feat = accf * fs_ref[...] + fb_ref[...]
    attn = acca + ba_ref[...]
    m = m_ref[0]
    attn = jnp.where(m > 0, attn, -1e9)
    amax = jnp.max(attn, axis=0, keepdims=True)
    e = jnp.exp(attn - amax)
    w = e / jnp.sum(e, axis=0, keepdims=True)
    wsum_ref[...] = jnp.sum(feat * w, axis=0, keepdims=True)[None]
    fm = jnp.where(m > 0, feat, F32_MIN)
    gmax_ref[...] = jnp.max(fm, axis=0, keepdims=True)[None]


def _tenc(x_pad, wf_r, wa_r, fs, fb, ba2, mask_f, ct):
    nt = H // ct
    return pl.pallas_call(
        _tenc_body,
        grid=(nt, B),
        in_specs=[
            pl.BlockSpec((1, LP, D_T), lambda t, b: (b, 0, 0)),
            pl.BlockSpec((KT * D_T, ct), lambda t, b: (0, t)),
            pl.BlockSpec((KT * D_T, ct), lambda t, b: (0, t)),
            pl.BlockSpec((1, ct), lambda t, b: (0, t)),
            pl.BlockSpec((1, ct), lambda t, b: (0, t)),
            pl.BlockSpec((1, ct), lambda t, b: (0, t)),
            pl.BlockSpec((1, L, 1), lambda t, b: (b, 0, 0)),
        ],
        out_specs=[
            pl.BlockSpec((1, 1, ct), lambda t, b: (b, 0, t)),
            pl.BlockSpec((1, 1, ct), lambda t, b: (b, 0, t)),
        ],
        out_shape=[
            jax.ShapeDtypeStruct((B, 1, H), jnp.float32),
            jax.ShapeDtypeStruct((B, 1, H), jnp.float32),
        ],
    )(x_pad, wf_r, wa_r, fs, fb, ba2, mask_f)


def _proj_body(de_ref, wd_ref, co_ref, wp_ref, dsc_ref, dbi_ref, psc_ref,
               pbi_ref, zd_ref, zt_ref):
    a = jnp.dot(de_ref[...], wd_ref[...], preferred_element_type=jnp.float32)
    zd_ref[...] = _leaky(a * dsc_ref[...] + dbi_ref[...])
    b = jnp.dot(co_ref[...], wp_ref[...], preferred_element_type=jnp.float32)
    zt_ref[...] = _leaky(b * psc_ref[...] + pbi_ref[...])


def _proj(de, wd, co, wp, dsc, dbi, psc, pbi):
    return pl.pallas_call(
        _proj_body,
        out_shape=[
            jax.ShapeDtypeStruct((B, OUT), jnp.float32),
            jax.ShapeDtypeStruct((B, OUT), jnp.float32),
        ],
    )(de, wd, co, wp, dsc, dbi, psc, pbi)


def _vq_body(zr_ref, cb_ref, idx_ref, commit_ref, perp_ref):
    n = zr_ref.shape[1]
    zr = zr_ref[0]
    zrb = zr.astype(jnp.bfloat16)
    run_m = jnp.full((n, 1), jnp.inf, jnp.float32)
    run_i = jnp.zeros((n, 1), jnp.int32)
    ones_row = jnp.ones((1, CODE), jnp.float32)
    for c in range(K_CB // CB_CH):
        cbc = cb_ref[0, c * CB_CH:(c + 1) * CB_CH, :]
        d = lax.dot_general(zrb, cbc, (((1,), (1,)), ((), ())),
                            preferred_element_type=jnp.float32)
        cbf = cbc.astype(jnp.float32)
        c2 = lax.dot_general(ones_row, cbf * cbf, (((1,), (1,)), ((), ())),
                             preferred_element_type=jnp.float32)
        s = c2 - 2.0 * d
        m = jnp.min(s, axis=1, keepdims=True)
        jg = lax.broadcasted_iota(jnp.int32, (n, CB_CH), 1) + c * CB_CH
        cand = jnp.min(jnp.where(s == m, jg, jnp.int32(2**31 - 1)),
                       axis=1, keepdims=True)
        better = m < run_m
        run_i = jnp.where(better, cand, run_i)
        run_m = jnp.where(better, m, run_m)
    idx_ref[0] = run_i
    z2 = jnp.sum(zr * zr)
    commit_ref[...] = ((z2 + jnp.sum(run_m)) / float(n * CODE)).reshape(1, 1, 1)
    plogp = jnp.zeros((), jnp.float32)
    for c in range(K_CB // CB_CH):
        bins = lax.broadcasted_iota(jnp.int32, (n, CB_CH), 1) + c * CB_CH
        cnt = jnp.sum((run_i == bins).astype(jnp.float32), axis=0,
                      keepdims=True)
        p = cnt / float(n)
        plogp += jnp.sum(p * jnp.log(p + 1e-10))
    perp_ref[...] = jnp.exp(-plogp).reshape(1, 1, 1)


def _vq(zr2, cb2):
    npath = zr2.shape[0]
    n = zr2.shape[1]
    return pl.pallas_call(
        _vq_body,
        grid=(npath,),
        in_specs=[
            pl.BlockSpec((1, n, CODE), lambda p: (p, 0, 0)),
            pl.BlockSpec((1, K_CB, CODE), lambda p: (p, 0, 0)),
        ],
        out_specs=[
            pl.BlockSpec((1, n, 1), lambda p: (p, 0, 0)),
            pl.BlockSpec((1, 1, 1), lambda p: (p, 0, 0)),
            pl.BlockSpec((1, 1, 1), lambda p: (p, 0, 0)),
        ],
        out_shape=[
            jax.ShapeDtypeStruct((npath, n, 1), jnp.int32),
            jax.ShapeDtypeStruct((npath, 1, 1), jnp.float32),
            jax.ShapeDtypeStruct((npath, 1, 1), jnp.float32),
        ],
    )(zr2, cb2)


_SC_ROWS = NVEC // 32


def _sc_gather_body(idx_hbm, cbd_hbm, cbt_hbm, q_hbm, idx_v, rows_v, sem):
    wid = lax.axis_index("s") * 2 + lax.axis_index("c")
    base = wid * _SC_ROWS
    pltpu.sync_copy(idx_hbm.at[pl.ds(base, _SC_ROWS)], idx_v)

    @pl.when(wid < 16)
    def _():
        pltpu.async_copy(cbd_hbm.at[idx_v], rows_v, sem).wait()

    @pl.when(wid >= 16)
    def _():
        pltpu.async_copy(cbt_hbm.at[idx_v], rows_v, sem).wait()

    pltpu.sync_copy(rows_v, q_hbm.at[pl.ds(base, _SC_ROWS)])


@functools.lru_cache(maxsize=1)
def _sc_gather_built():
    return pl.kernel(
        _sc_gather_body,
        mesh=plsc.VectorSubcoreMesh(core_axis_name="c", subcore_axis_name="s"),
        out_type=jax.ShapeDtypeStruct((NVEC, 128), jnp.float32),
        scratch_types=[
            pltpu.VMEM((_SC_ROWS,), jnp.int32),
            pltpu.VMEM((_SC_ROWS, 128), jnp.float32),
            pltpu.SemaphoreType.DMA,
        ],
    )


def _sc_gather(idx_all, cbd, cbt):
    return _sc_gather_built()(idx_all, cbd, cbt)


def _digamma1(x):
    acc = jnp.zeros_like(x)
    for k in range(8):
        acc += 1.0 / (x + float(k))
    y = x + 8.0
    inv = 1.0 / y
    inv2 = inv * inv
    tail = inv2 * (1.0 / 12.0 - inv2 * (1.0 / 120.0 - inv2 * (1.0 / 252.0)))
    return jnp.log(y) - 0.5 * inv - tail - acc


def _gammaln1(x):
    acc = jnp.zeros_like(x)
    for k in range(8):
        acc += jnp.log(x + float(k))
    y = x + 8.0
    inv = 1.0 / y
    inv2 = inv * inv
    tail = inv * (1.0 / 12.0 - inv2 * (1.0 / 360.0 - inv2 * (1.0 / 1260.0)))
    return (y - 0.5) * jnp.log(y) - y + 0.9189385332046727 + tail - acc


_LGAMMA_NC = math.lgamma(float(NC))


def _head_body(fu_ref, we1_ref, esc_ref, ebi_ref, we2_ref, be2_ref, y_ref,
               cm_ref, out_ref):
    h = jnp.dot(fu_ref[...], we1_ref[...], preferred_element_type=jnp.float32)
    h = _leaky(h * esc_ref[...] + ebi_ref[...])
    logits = jnp.dot(h, we2_ref[...], preferred_element_type=jnp.float32)
    logits = logits + be2_ref[...]
    col = lax.broadcasted_iota(jnp.int32, (B, 128), 1)
    valid = col < NC
    sp = jnp.maximum(logits, 0.0) + jnp.log1p(jnp.exp(-jnp.abs(logits)))
    alpha = sp + 1.0
    alpha_s = jnp.where(valid, alpha, 1.0)
    y_oh = (col == y_ref[...]).astype(jnp.float32)
    S = jnp.sum(jnp.where(valid, alpha, 0.0), axis=1, keepdims=True)
    err = jnp.sum(y_oh * (_digamma1(S) - _digamma1(alpha_s)), axis=1,
                  keepdims=True)
    a_t = jnp.where(valid, y_oh + (1.0 - y_oh) * alpha_s, 1.0)
    St = jnp.sum(jnp.where(valid, a_t, 0.0), axis=1, keepdims=True)
    kl = (_gammaln1(St)
          - jnp.sum(jnp.where(valid, _gammaln1(a_t), 0.0), axis=1,
                    keepdims=True)
          - _LGAMMA_NC
          + jnp.sum(jnp.where(valid, (a_t - 1.0)
                              * (_digamma1(a_t) - _digamma1(St)), 0.0),
                    axis=1, keepdims=True))
    class_loss = jnp.sum(err + 0.1 * kl) / float(B)
    loss = class_loss + cm_ref[0, 0, 0] + cm_ref[1, 0, 0]
    out_ref[...] = jnp.where(col[0:1, :] == 0, loss, class_loss)


def _head(fused, we1, esc, ebi, we2p, be2p, y2, commit):
    return pl.pallas_call(
        _head_body,
        in_specs=[
            pl.BlockSpec((B, 2 * OUT), lambda: (0, 0)),
            pl.BlockSpec((2 * OUT, 512), lambda: (0, 0)),
            pl.BlockSpec((1, 512), lambda: (0, 0)),
            pl.BlockSpec((1, 512), lambda: (0, 0)),
            pl.BlockSpec((512, 128), lambda: (0, 0)),
            pl.BlockSpec((1, 128), lambda: (0, 0)),
            pl.BlockSpec((B, 1), lambda: (0, 0)),
            pl.BlockSpec(memory_space=pltpu.SMEM),
        ],
        out_shape=jax.ShapeDtypeStruct((1, 128), jnp.float32),
    )(fused, we1, esc, ebi, we2p, be2p, y2, commit)


def kernel(d_emb, t_emb, t_mask, y, W_d, b_d, dg, db, Wf, bf, fg, fb, Wa, ba,
           Wp, bp, pg, pb, cb_d, cb_t, We1, be1, eg, eb, We2, be2):
    f32 = jnp.float32
    bf16 = jnp.bfloat16

    x_pad = jnp.pad(t_emb, ((0, 0), (KT // 2, KT // 2), (0, 0))).astype(bf16)
    wf_r = jnp.transpose(Wf, (2, 1, 0)).astype(bf16).reshape(KT * D_T, H)
    wa_r = jnp.transpose(Wa, (2, 1, 0)).astype(bf16).reshape(KT * D_T, H)
    fs = (fg * RSQ).reshape(1, H)
    fbv = (bf * fg * RSQ + fb).reshape(1, H)
    ba2 = ba.reshape(1, H)
    mask_f = t_mask.astype(f32).reshape(B, L, 1)
    wsum, gmax = _tenc(x_pad, wf_r, wa_r, fs, fbv, ba2, mask_f, ct=H)
    comb = jnp.concatenate([wsum.reshape(B, H), gmax.reshape(B, H)],
                           axis=-1).astype(bf16)

    dsc = (dg * RSQ).reshape(1, OUT)
    dbi = (b_d * dg * RSQ + db).reshape(1, OUT)
    psc = (pg * RSQ).reshape(1, OUT)
    pbi = (bp * pg * RSQ + pb).reshape(1, OUT)
    z_d, z_t = _proj(d_emb.astype(bf16), W_d.astype(bf16), comb,
                     Wp.astype(bf16), dsc, dbi, psc, pbi)

    zr2 = jnp.stack([z_d.reshape(-1, CODE), z_t.reshape(-1, CODE)])
    cb2 = jnp.stack([cb_d, cb_t]).astype(bf16)
    idx2, commit2, perp2 = _vq(zr2, cb2)

    idx_all = idx2.reshape(NVEC)
    cbd_p = jnp.pad(cb_d, ((0, 0), (0, 128 - CODE)))
    cbt_p = jnp.pad(cb_t, ((0, 0), (0, 128 - CODE)))
    q_all = _sc_gather(idx_all, cbd_p, cbt_p)[:, :CODE]
    q_d = q_all[:NVEC // 2].reshape(B, OUT)
    q_t = q_all[NVEC // 2:].reshape(B, OUT)

    fused = jnp.concatenate([q_d, q_t], axis=-1)
    esc = (eg * RSQ).reshape(1, 512)
    ebi = (be1 * eg * RSQ + eb).reshape(1, 512)
    we2p = jnp.pad(We2, ((0, 0), (0, 128 - NC)))
    be2p = jnp.pad(be2, (0, 128 - NC)).reshape(1, 128)
    y2 = y.astype(jnp.int32).reshape(B, 1)
    hrow = _head(fused, We1, esc, ebi, we2p, be2p, y2, commit2)

    loss = hrow[0, 0]
    class_loss = hrow[0, 1]
    d_vq_loss = commit2[0, 0, 0]
    t_vq_loss = commit2[1, 0, 0]
    d_perp = perp2[0, 0, 0]
    t_perp = perp2[1, 0, 0]
    return (loss, class_loss, d_vq_loss, t_vq_loss, d_perp, t_perp)


# speedup vs baseline: 1.3837x; 1.0808x over previous
import functools
import math

import jax
import jax.numpy as jnp
from jax import lax
from jax.experimental import pallas as pl
from jax.experimental.pallas import tpu as pltpu
from jax.experimental.pallas import tpu_sc as plsc

B = 64; L = 256; D_T = 1280; D_D = 2048; H = 768; OUT = 1024
K_CB = 8192; CODE = 64; NC = 5
KT = 9
LP = L + KT - 1
RSQ = 1.0 / math.sqrt(1.0 + 1e-05)
NVEC = (2 * B * OUT) // CODE
CB_CH = 1024
F32_MIN = float(jnp.finfo(jnp.float32).min)


def _leaky(x):
    return jnp.where(x >= 0, x, 0.01 * x)


def _tenc_body(x_ref, wf_ref, wa_ref, fs_ref, fb_ref, ba_ref, m_ref,
               wsum_ref, gmax_ref):
    xf = x_ref[0].astype(jnp.bfloat16)
    zpad = jnp.zeros((KT // 2, D_T), jnp.bfloat16)
    x = jnp.concatenate([zpad, xf, zpad], axis=0)
    ct = wf_ref.shape[1]
    accf = jnp.zeros((L, ct), jnp.float32)
    acca = jnp.zeros((L, ct), jnp.float32)
    for k in range(KT):
        xs = x[k:k + L, :]
        accf += jnp.dot(xs, wf_ref[k * D_T:(k + 1) * D_T, :],
                        preferred_element_type=jnp.float32)
        acca += jnp.dot(xs, wa_ref[k * D_T:(k + 1) * D_T, :],
                        preferred_element_type=jnp.float32)
    feat = accf * fs_ref[...] + fb_ref[...]
    attn = acca + ba_ref[...]
    m = m_ref[0]
    attn = jnp.where(m > 0, attn, -1e9)
    amax = jnp.max(attn, axis=0, keepdims=True)
    e = jnp.exp(attn - amax)
    w = e / jnp.sum(e, axis=0, keepdims=True)
    wsum_ref[...] = jnp.sum(feat * w, axis=0, keepdims=True)[None]
    fm = jnp.where(m > 0, feat, F32_MIN)
    gmax_ref[...] = jnp.max(fm, axis=0, keepdims=True)[None]


def _tenc(x_pad, wf_r, wa_r, fs, fb, ba2, mask_f, ct):
    nt = H // ct
    return pl.pallas_call(
        _tenc_body,
        grid=(nt, B),
        in_specs=[
            pl.BlockSpec((1, L, D_T), lambda t, b: (b, 0, 0)),
            pl.BlockSpec((KT * D_T, ct), lambda t, b: (0, t)),
            pl.BlockSpec((KT * D_T, ct), lambda t, b: (0, t)),
            pl.BlockSpec((1, ct), lambda t, b: (0, t)),
            pl.BlockSpec((1, ct), lambda t, b: (0, t)),
            pl.BlockSpec((1, ct), lambda t, b: (0, t)),
            pl.BlockSpec((1, L, 1), lambda t, b: (b, 0, 0)),
        ],
        out_specs=[
            pl.BlockSpec((1, 1, ct), lambda t, b: (b, 0, t)),
            pl.BlockSpec((1, 1, ct), lambda t, b: (b, 0, t)),
        ],
        out_shape=[
            jax.ShapeDtypeStruct((B, 1, H), jnp.float32),
            jax.ShapeDtypeStruct((B, 1, H), jnp.float32),
        ],
    )(x_pad, wf_r, wa_r, fs, fb, ba2, mask_f)


def _proj_body(de_ref, wd_ref, co_ref, wp_ref, dsc_ref, dbi_ref, psc_ref,
               pbi_ref, zd_ref, zt_ref):
    a = jnp.dot(de_ref[...], wd_ref[...], preferred_element_type=jnp.float32)
    zd_ref[...] = _leaky(a * dsc_ref[...] + dbi_ref[...])
    b = jnp.dot(co_ref[...], wp_ref[...], preferred_element_type=jnp.float32)
    zt_ref[...] = _leaky(b * psc_ref[...] + pbi_ref[...])


def _proj(de, wd, co, wp, dsc, dbi, psc, pbi):
    return pl.pallas_call(
        _proj_body,
        out_shape=[
            jax.ShapeDtypeStruct((B, OUT), jnp.float32),
            jax.ShapeDtypeStruct((B, OUT), jnp.float32),
        ],
    )(de, wd, co, wp, dsc, dbi, psc, pbi)


def _vq_body(zr_ref, cb_ref, idx_ref, commit_ref, perp_ref):
    n = zr_ref.shape[1]
    zr = zr_ref[0]
    zrb = zr.astype(jnp.bfloat16)
    run_m = jnp.full((n, 1), jnp.inf, jnp.float32)
    run_i = jnp.zeros((n, 1), jnp.int32)
    ones_row = jnp.ones((1, CODE), jnp.float32)
    for c in range(K_CB // CB_CH):
        cbc = cb_ref[0, c * CB_CH:(c + 1) * CB_CH, :]
        d = lax.dot_general(zrb, cbc, (((1,), (1,)), ((), ())),
                            preferred_element_type=jnp.float32)
        cbf = cbc.astype(jnp.float32)
        c2 = lax.dot_general(ones_row, cbf * cbf, (((1,), (1,)), ((), ())),
                             preferred_element_type=jnp.float32)
        s = c2 - 2.0 * d
        m = jnp.min(s, axis=1, keepdims=True)
        jg = lax.broadcasted_iota(jnp.int32, (n, CB_CH), 1) + c * CB_CH
        cand = jnp.min(jnp.where(s == m, jg, jnp.int32(2**31 - 1)),
                       axis=1, keepdims=True)
        better = m < run_m
        run_i = jnp.where(better, cand, run_i)
        run_m = jnp.where(better, m, run_m)
    idx_ref[0] = run_i
    z2 = jnp.sum(zr * zr)
    commit_ref[...] = ((z2 + jnp.sum(run_m)) / float(n * CODE)).reshape(1, 1, 1)
    plogp = jnp.zeros((), jnp.float32)
    for c in range(K_CB // CB_CH):
        bins = lax.broadcasted_iota(jnp.int32, (n, CB_CH), 1) + c * CB_CH
        cnt = jnp.sum((run_i == bins).astype(jnp.float32), axis=0,
                      keepdims=True)
        p = cnt / float(n)
        plogp += jnp.sum(p * jnp.log(p + 1e-10))
    perp_ref[...] = jnp.exp(-plogp).reshape(1, 1, 1)


def _vq(zr2, cb2):
    npath = zr2.shape[0]
    n = zr2.shape[1]
    return pl.pallas_call(
        _vq_body,
        grid=(npath,),
        in_specs=[
            pl.BlockSpec((1, n, CODE), lambda p: (p, 0, 0)),
            pl.BlockSpec((1, K_CB, CODE), lambda p: (p, 0, 0)),
        ],
        out_specs=[
            pl.BlockSpec((1, n, 1), lambda p: (p, 0, 0)),
            pl.BlockSpec((1, 1, 1), lambda p: (p, 0, 0)),
            pl.BlockSpec((1, 1, 1), lambda p: (p, 0, 0)),
        ],
        out_shape=[
            jax.ShapeDtypeStruct((npath, n, 1), jnp.int32),
            jax.ShapeDtypeStruct((npath, 1, 1), jnp.float32),
            jax.ShapeDtypeStruct((npath, 1, 1), jnp.float32),
        ],
    )(zr2, cb2)


_SC_ROWS = NVEC // 32


def _sc_gather_body(idx_hbm, cbd_hbm, cbt_hbm, q_hbm, idx_v, rows_v, sem):
    wid = lax.axis_index("s") * 2 + lax.axis_index("c")
    base = wid * _SC_ROWS
    pltpu.sync_copy(idx_hbm.at[pl.ds(base, _SC_ROWS)], idx_v)

    @pl.when(wid < 16)
    def _():
        pltpu.async_copy(cbd_hbm.at[idx_v], rows_v, sem).wait()

    @pl.when(wid >= 16)
    def _():
        pltpu.async_copy(cbt_hbm.at[idx_v], rows_v, sem).wait()

    pltpu.sync_copy(rows_v, q_hbm.at[pl.ds(base, _SC_ROWS)])


@functools.lru_cache(maxsize=1)
def _sc_gather_built():
    return pl.kernel(
        _sc_gather_body,
        mesh=plsc.VectorSubcoreMesh(core_axis_name="c", subcore_axis_name="s"),
        out_type=jax.ShapeDtypeStruct((NVEC, 128), jnp.float32),
        scratch_types=[
            pltpu.VMEM((_SC_ROWS,), jnp.int32),
            pltpu.VMEM((_SC_ROWS, 128), jnp.float32),
            pltpu.SemaphoreType.DMA,
        ],
    )


def _sc_gather(idx_all, cbd, cbt):
    return _sc_gather_built()(idx_all, cbd, cbt)


def _digamma1(x):
    acc = jnp.zeros_like(x)
    for k in range(8):
        acc += 1.0 / (x + float(k))
    y = x + 8.0
    inv = 1.0 / y
    inv2 = inv * inv
    tail = inv2 * (1.0 / 12.0 - inv2 * (1.0 / 120.0 - inv2 * (1.0 / 252.0)))
    return jnp.log(y) - 0.5 * inv - tail - acc


def _gammaln1(x):
    acc = jnp.zeros_like(x)
    for k in range(8):
        acc += jnp.log(x + float(k))
    y = x + 8.0
    inv = 1.0 / y
    inv2 = inv * inv
    tail = inv * (1.0 / 12.0 - inv2 * (1.0 / 360.0 - inv2 * (1.0 / 1260.0)))
    return (y - 0.5) * jnp.log(y) - y + 0.9189385332046727 + tail - acc


_LGAMMA_NC = math.lgamma(float(NC))


def _head_body(fu_ref, we1_ref, esc_ref, ebi_ref, we2_ref, be2_ref, y_ref,
               cm_ref, out_ref):
    h = jnp.dot(fu_ref[...], we1_ref[...], preferred_element_type=jnp.float32)
    h = _leaky(h * esc_ref[...] + ebi_ref[...])
    logits = jnp.dot(h, we2_ref[...], preferred_element_type=jnp.float32)
    logits = logits + be2_ref[...]
    col = lax.broadcasted_iota(jnp.int32, (B, 128), 1)
    valid = col < NC
    sp = jnp.maximum(logits, 0.0) + jnp.log1p(jnp.exp(-jnp.abs(logits)))
    alpha = sp + 1.0
    alpha_s = jnp.where(valid, alpha, 1.0)
    y_oh = (col == y_ref[...]).astype(jnp.float32)
    S = jnp.sum(jnp.where(valid, alpha, 0.0), axis=1, keepdims=True)
    err = jnp.sum(y_oh * (_digamma1(S) - _digamma1(alpha_s)), axis=1,
                  keepdims=True)
    a_t = jnp.where(valid, y_oh + (1.0 - y_oh) * alpha_s, 1.0)
    St = jnp.sum(jnp.where(valid, a_t, 0.0), axis=1, keepdims=True)
    kl = (_gammaln1(St)
          - jnp.sum(jnp.where(valid, _gammaln1(a_t), 0.0), axis=1,
                    keepdims=True)
          - _LGAMMA_NC
          + jnp.sum(jnp.where(valid, (a_t - 1.0)
                              * (_digamma1(a_t) - _digamma1(St)), 0.0),
                    axis=1, keepdims=True))
    class_loss = jnp.sum(err + 0.1 * kl) / float(B)
    loss = class_loss + cm_ref[0, 0, 0] + cm_ref[1, 0, 0]
    out_ref[...] = jnp.where(col[0:1, :] == 0, loss, class_loss)


def _head(fused, we1, esc, ebi, we2p, be2p, y2, commit):
    return pl.pallas_call(
        _head_body,
        in_specs=[
            pl.BlockSpec((B, 2 * OUT), lambda: (0, 0)),
            pl.BlockSpec((2 * OUT, 512), lambda: (0, 0)),
            pl.BlockSpec((1, 512), lambda: (0, 0)),
            pl.BlockSpec((1, 512), lambda: (0, 0)),
            pl.BlockSpec((512, 128), lambda: (0, 0)),
            pl.BlockSpec((1, 128), lambda: (0, 0)),
            pl.BlockSpec((B, 1), lambda: (0, 0)),
            pl.BlockSpec(memory_space=pltpu.SMEM),
        ],
        out_shape=jax.ShapeDtypeStruct((1, 128), jnp.float32),
    )(fused, we1, esc, ebi, we2p, be2p, y2, commit)


def kernel(d_emb, t_emb, t_mask, y, W_d, b_d, dg, db, Wf, bf, fg, fb, Wa, ba,
           Wp, bp, pg, pb, cb_d, cb_t, We1, be1, eg, eb, We2, be2):
    f32 = jnp.float32
    bf16 = jnp.bfloat16

    wf_r = jnp.transpose(Wf, (2, 1, 0)).astype(bf16).reshape(KT * D_T, H)
    wa_r = jnp.transpose(Wa, (2, 1, 0)).astype(bf16).reshape(KT * D_T, H)
    fs = (fg * RSQ).reshape(1, H)
    fbv = (bf * fg * RSQ + fb).reshape(1, H)
    ba2 = ba.reshape(1, H)
    mask_f = t_mask.astype(f32).reshape(B, L, 1)
    wsum, gmax = _tenc(t_emb, wf_r, wa_r, fs, fbv, ba2, mask_f, ct=H)
    comb = jnp.concatenate([wsum.reshape(B, H), gmax.reshape(B, H)],
                           axis=-1).astype(bf16)

    dsc = (dg * RSQ).reshape(1, OUT)
    dbi = (b_d * dg * RSQ + db).reshape(1, OUT)
    psc = (pg * RSQ).reshape(1, OUT)
    pbi = (bp * pg * RSQ + pb).reshape(1, OUT)
    z_d, z_t = _proj(d_emb.astype(bf16), W_d.astype(bf16), comb,
                     Wp.astype(bf16), dsc, dbi, psc, pbi)

    zr2 = jnp.stack([z_d.reshape(-1, CODE), z_t.reshape(-1, CODE)])
    cb2 = jnp.stack([cb_d, cb_t]).astype(bf16)
    idx2, commit2, perp2 = _vq(zr2, cb2)

    idx_all = idx2.reshape(NVEC)
    cbd_p = jnp.pad(cb_d, ((0, 0), (0, 128 - CODE)))
    cbt_p = jnp.pad(cb_t, ((0, 0), (0, 128 - CODE)))
    q_all = _sc_gather(idx_all, cbd_p, cbt_p)[:, :CODE]
    q_d = q_all[:NVEC // 2].reshape(B, OUT)
    q_t = q_all[NVEC // 2:].reshape(B, OUT)

    fused = jnp.concatenate([q_d, q_t], axis=-1)
    esc = (eg * RSQ).reshape(1, 512)
    ebi = (be1 * eg * RSQ + eb).reshape(1, 512)
    we2p = jnp.pad(We2, ((0, 0), (0, 128 - NC)))
    be2p = jnp.pad(be2, (0, 128 - NC)).reshape(1, 128)
    y2 = y.astype(jnp.int32).reshape(B, 1)
    hrow = _head(fused, We1, esc, ebi, we2p, be2p, y2, commit2)

    loss = hrow[0, 0]
    class_loss = hrow[0, 1]
    d_vq_loss = commit2[0, 0, 0]
    t_vq_loss = commit2[1, 0, 0]
    d_perp = perp2[0, 0, 0]
    t_perp = perp2[1, 0, 0]
    return (loss, class_loss, d_vq_loss, t_vq_loss, d_perp, t_perp)


# speedup vs baseline: 1.3847x; 1.0007x over previous
import functools
import math

import jax
import jax.numpy as jnp
from jax import lax
from jax.experimental import pallas as pl
from jax.experimental.pallas import tpu as pltpu
from jax.experimental.pallas import tpu_sc as plsc

B = 64; L = 256; D_T = 1280; D_D = 2048; H = 768; OUT = 1024
K_CB = 8192; CODE = 64; NC = 5
KT = 9
LP = L + KT - 1
RSQ = 1.0 / math.sqrt(1.0 + 1e-05)
NVEC = (2 * B * OUT) // CODE
CB_CH = 1024
F32_MIN = float(jnp.finfo(jnp.float32).min)


def _leaky(x):
    return jnp.where(x >= 0, x, 0.01 * x)


def _tenc_body(x_ref, wf_ref, wa_ref, fs_ref, fb_ref, ba_ref, m_ref,
               wsum_ref, gmax_ref):
    xf = x_ref[0].astype(jnp.bfloat16)
    zpad = jnp.zeros((KT // 2, D_T), jnp.bfloat16)
    x = jnp.concatenate([zpad, xf, zpad], axis=0)
    patches = jnp.concatenate([x[k:k + L, :] for k in range(KT)], axis=1)
    accf = jnp.dot(patches, wf_ref[...], preferred_element_type=jnp.float32)
    acca = jnp.dot(patches, wa_ref[...], preferred_element_type=jnp.float32)
    feat = accf * fs_ref[...] + fb_ref[...]
    attn = acca + ba_ref[...]
    m = m_ref[0]
    attn = jnp.where(m > 0, attn, -1e9)
    amax = jnp.max(attn, axis=0, keepdims=True)
    e = jnp.exp(attn - amax)
    w = e / jnp.sum(e, axis=0, keepdims=True)
    wsum_ref[...] = jnp.sum(feat * w, axis=0, keepdims=True)[None]
    fm = jnp.where(m > 0, feat, F32_MIN)
    gmax_ref[...] = jnp.max(fm, axis=0, keepdims=True)[None]


def _tenc(x_pad, wf_r, wa_r, fs, fb, ba2, mask_f, ct):
    nt = H // ct
    return pl.pallas_call(
        _tenc_body,
        grid=(nt, B),
        in_specs=[
            pl.BlockSpec((1, L, D_T), lambda t, b: (b, 0, 0)),
            pl.BlockSpec((KT * D_T, ct), lambda t, b: (0, t)),
            pl.BlockSpec((KT * D_T, ct), lambda t, b: (0, t)),
            pl.BlockSpec((1, ct), lambda t, b: (0, t)),
            pl.BlockSpec((1, ct), lambda t, b: (0, t)),
            pl.BlockSpec((1, ct), lambda t, b: (0, t)),
            pl.BlockSpec((1, L, 1), lambda t, b: (b, 0, 0)),
        ],
        out_specs=[
            pl.BlockSpec((1, 1, ct), lambda t, b: (b, 0, t)),
            pl.BlockSpec((1, 1, ct), lambda t, b: (b, 0, t)),
        ],
        out_shape=[
            jax.ShapeDtypeStruct((B, 1, H), jnp.float32),
            jax.ShapeDtypeStruct((B, 1, H), jnp.float32),
        ],
    )(x_pad, wf_r, wa_r, fs, fb, ba2, mask_f)


def _proj_body(de_ref, wd_ref, co_ref, wp_ref, dsc_ref, dbi_ref, psc_ref,
               pbi_ref, zd_ref, zt_ref):
    a = jnp.dot(de_ref[...], wd_ref[...], preferred_element_type=jnp.float32)
    zd_ref[...] = _leaky(a * dsc_ref[...] + dbi_ref[...])
    b = jnp.dot(co_ref[...], wp_ref[...], preferred_element_type=jnp.float32)
    zt_ref[...] = _leaky(b * psc_ref[...] + pbi_ref[...])


def _proj(de, wd, co, wp, dsc, dbi, psc, pbi):
    return pl.pallas_call(
        _proj_body,
        out_shape=[
            jax.ShapeDtypeStruct((B, OUT), jnp.float32),
            jax.ShapeDtypeStruct((B, OUT), jnp.float32),
        ],
    )(de, wd, co, wp, dsc, dbi, psc, pbi)


def _vq_body(zr_ref, cb_ref, idx_ref, commit_ref, perp_ref):
    n = zr_ref.shape[1]
    zr = zr_ref[0]
    zrb = zr.astype(jnp.bfloat16)
    run_m = jnp.full((n, 1), jnp.inf, jnp.float32)
    run_i = jnp.zeros((n, 1), jnp.int32)
    ones_row = jnp.ones((1, CODE), jnp.float32)
    for c in range(K_CB // CB_CH):
        cbc = cb_ref[0, c * CB_CH:(c + 1) * CB_CH, :]
        d = lax.dot_general(zrb, cbc, (((1,), (1,)), ((), ())),
                            preferred_element_type=jnp.float32)
        cbf = cbc.astype(jnp.float32)
        c2 = lax.dot_general(ones_row, cbf * cbf, (((1,), (1,)), ((), ())),
                             preferred_element_type=jnp.float32)
        s = c2 - 2.0 * d
        m = jnp.min(s, axis=1, keepdims=True)
        jg = lax.broadcasted_iota(jnp.int32, (n, CB_CH), 1) + c * CB_CH
        cand = jnp.min(jnp.where(s == m, jg, jnp.int32(2**31 - 1)),
                       axis=1, keepdims=True)
        better = m < run_m
        run_i = jnp.where(better, cand, run_i)
        run_m = jnp.where(better, m, run_m)
    idx_ref[0] = run_i
    z2 = jnp.sum(zr * zr)
    commit_ref[...] = ((z2 + jnp.sum(run_m)) / float(n * CODE)).reshape(1, 1, 1)
    plogp = jnp.zeros((), jnp.float32)
    for c in range(K_CB // CB_CH):
        bins = lax.broadcasted_iota(jnp.int32, (n, CB_CH), 1) + c * CB_CH
        cnt = jnp.sum((run_i == bins).astype(jnp.float32), axis=0,
                      keepdims=True)
        p = cnt / float(n)
        plogp += jnp.sum(p * jnp.log(p + 1e-10))
    perp_ref[...] = jnp.exp(-plogp).reshape(1, 1, 1)


def _vq(zr2, cb2):
    npath = zr2.shape[0]
    n = zr2.shape[1]
    return pl.pallas_call(
        _vq_body,
        grid=(npath,),
        in_specs=[
            pl.BlockSpec((1, n, CODE), lambda p: (p, 0, 0)),
            pl.BlockSpec((1, K_CB, CODE), lambda p: (p, 0, 0)),
        ],
        out_specs=[
            pl.BlockSpec((1, n, 1), lambda p: (p, 0, 0)),
            pl.BlockSpec((1, 1, 1), lambda p: (p, 0, 0)),
            pl.BlockSpec((1, 1, 1), lambda p: (p, 0, 0)),
        ],
        out_shape=[
            jax.ShapeDtypeStruct((npath, n, 1), jnp.int32),
            jax.ShapeDtypeStruct((npath, 1, 1), jnp.float32),
            jax.ShapeDtypeStruct((npath, 1, 1), jnp.float32),
        ],
    )(zr2, cb2)


_SC_ROWS = NVEC // 32


def _sc_gather_body(idx_hbm, cbd_hbm, cbt_hbm, q_hbm, idx_v, rows_v, sem):
    wid = lax.axis_index("s") * 2 + lax.axis_index("c")
    base = wid * _SC_ROWS
    pltpu.sync_copy(idx_hbm.at[pl.ds(base, _SC_ROWS)], idx_v)

    @pl.when(wid < 16)
    def _():
        pltpu.async_copy(cbd_hbm.at[idx_v], rows_v, sem).wait()

    @pl.when(wid >= 16)
    def _():
        pltpu.async_copy(cbt_hbm.at[idx_v], rows_v, sem).wait()

    pltpu.sync_copy(rows_v, q_hbm.at[pl.ds(base, _SC_ROWS)])


@functools.lru_cache(maxsize=1)
def _sc_gather_built():
    return pl.kernel(
        _sc_gather_body,
        mesh=plsc.VectorSubcoreMesh(core_axis_name="c", subcore_axis_name="s"),
        out_type=jax.ShapeDtypeStruct((NVEC, 128), jnp.float32),
        scratch_types=[
            pltpu.VMEM((_SC_ROWS,), jnp.int32),
            pltpu.VMEM((_SC_ROWS, 128), jnp.float32),
            pltpu.SemaphoreType.DMA,
        ],
    )


def _sc_gather(idx_all, cbd, cbt):
    return _sc_gather_built()(idx_all, cbd, cbt)


def _digamma1(x):
    acc = jnp.zeros_like(x)
    for k in range(8):
        acc += 1.0 / (x + float(k))
    y = x + 8.0
    inv = 1.0 / y
    inv2 = inv * inv
    tail = inv2 * (1.0 / 12.0 - inv2 * (1.0 / 120.0 - inv2 * (1.0 / 252.0)))
    return jnp.log(y) - 0.5 * inv - tail - acc


def _gammaln1(x):
    acc = jnp.zeros_like(x)
    for k in range(8):
        acc += jnp.log(x + float(k))
    y = x + 8.0
    inv = 1.0 / y
    inv2 = inv * inv
    tail = inv * (1.0 / 12.0 - inv2 * (1.0 / 360.0 - inv2 * (1.0 / 1260.0)))
    return (y - 0.5) * jnp.log(y) - y + 0.9189385332046727 + tail - acc


_LGAMMA_NC = math.lgamma(float(NC))


def _head_body(fu_ref, we1_ref, esc_ref, ebi_ref, we2_ref, be2_ref, y_ref,
               cm_ref, out_ref):
    h = jnp.dot(fu_ref[...], we1_ref[...], preferred_element_type=jnp.float32)
    h = _leaky(h * esc_ref[...] + ebi_ref[...])
    logits = jnp.dot(h, we2_ref[...], preferred_element_type=jnp.float32)
    logits = logits + be2_ref[...]
    col = lax.broadcasted_iota(jnp.int32, (B, 128), 1)
    valid = col < NC
    sp = jnp.maximum(logits, 0.0) + jnp.log1p(jnp.exp(-jnp.abs(logits)))
    alpha = sp + 1.0
    alpha_s = jnp.where(valid, alpha, 1.0)
    y_oh = (col == y_ref[...]).astype(jnp.float32)
    S = jnp.sum(jnp.where(valid, alpha, 0.0), axis=1, keepdims=True)
    err = jnp.sum(y_oh * (_digamma1(S) - _digamma1(alpha_s)), axis=1,
                  keepdims=True)
    a_t = jnp.where(valid, y_oh + (1.0 - y_oh) * alpha_s, 1.0)
    St = jnp.sum(jnp.where(valid, a_t, 0.0), axis=1, keepdims=True)
    kl = (_gammaln1(St)
          - jnp.sum(jnp.where(valid, _gammaln1(a_t), 0.0), axis=1,
                    keepdims=True)
          - _LGAMMA_NC
          + jnp.sum(jnp.where(valid, (a_t - 1.0)
                              * (_digamma1(a_t) - _digamma1(St)), 0.0),
                    axis=1, keepdims=True))
    class_loss = jnp.sum(err + 0.1 * kl) / float(B)
    loss = class_loss + cm_ref[0, 0, 0] + cm_ref[1, 0, 0]
    out_ref[...] = jnp.where(col[0:1, :] == 0, loss, class_loss)


def _head(fused, we1, esc, ebi, we2p, be2p, y2, commit):
    return pl.pallas_call(
        _head_body,
        in_specs=[
            pl.BlockSpec((B, 2 * OUT), lambda: (0, 0)),
            pl.BlockSpec((2 * OUT, 512), lambda: (0, 0)),
            pl.BlockSpec((1, 512), lambda: (0, 0)),
            pl.BlockSpec((1, 512), lambda: (0, 0)),
            pl.BlockSpec((512, 128), lambda: (0, 0)),
            pl.BlockSpec((1, 128), lambda: (0, 0)),
            pl.BlockSpec((B, 1), lambda: (0, 0)),
            pl.BlockSpec(memory_space=pltpu.SMEM),
        ],
        out_shape=jax.ShapeDtypeStruct((1, 128), jnp.float32),
    )(fused, we1, esc, ebi, we2p, be2p, y2, commit)


def kernel(d_emb, t_emb, t_mask, y, W_d, b_d, dg, db, Wf, bf, fg, fb, Wa, ba,
           Wp, bp, pg, pb, cb_d, cb_t, We1, be1, eg, eb, We2, be2):
    f32 = jnp.float32
    bf16 = jnp.bfloat16

    wf_r = jnp.transpose(Wf, (2, 1, 0)).astype(bf16).reshape(KT * D_T, H)
    wa_r = jnp.transpose(Wa, (2, 1, 0)).astype(bf16).reshape(KT * D_T, H)
    fs = (fg * RSQ).reshape(1, H)
    fbv = (bf * fg * RSQ + fb).reshape(1, H)
    ba2 = ba.reshape(1, H)
    mask_f = t_mask.astype(f32).reshape(B, L, 1)
    wsum, gmax = _tenc(t_emb, wf_r, wa_r, fs, fbv, ba2, mask_f, ct=H)
    comb = jnp.concatenate([wsum.reshape(B, H), gmax.reshape(B, H)],
                           axis=-1).astype(bf16)

    dsc = (dg * RSQ).reshape(1, OUT)
    dbi = (b_d * dg * RSQ + db).reshape(1, OUT)
    psc = (pg * RSQ).reshape(1, OUT)
    pbi = (bp * pg * RSQ + pb).reshape(1, OUT)
    z_d, z_t = _proj(d_emb.astype(bf16), W_d.astype(bf16), comb,
                     Wp.astype(bf16), dsc, dbi, psc, pbi)

    zr2 = jnp.stack([z_d.reshape(-1, CODE), z_t.reshape(-1, CODE)])
    cb2 = jnp.stack([cb_d, cb_t]).astype(bf16)
    idx2, commit2, perp2 = _vq(zr2, cb2)

    idx_all = idx2.reshape(NVEC)
    cbd_p = jnp.pad(cb_d, ((0, 0), (0, 128 - CODE)))
    cbt_p = jnp.pad(cb_t, ((0, 0), (0, 128 - CODE)))
    q_all = _sc_gather(idx_all, cbd_p, cbt_p)[:, :CODE]
    q_d = q_all[:NVEC // 2].reshape(B, OUT)
    q_t = q_all[NVEC // 2:].reshape(B, OUT)

    fused = jnp.concatenate([q_d, q_t], axis=-1)
    esc = (eg * RSQ).reshape(1, 512)
    ebi = (be1 * eg * RSQ + eb).reshape(1, 512)
    we2p = jnp.pad(We2, ((0, 0), (0, 128 - NC)))
    be2p = jnp.pad(be2, (0, 128 - NC)).reshape(1, 128)
    y2 = y.astype(jnp.int32).reshape(B, 1)
    hrow = _head(fused, We1, esc, ebi, we2p, be2p, y2, commit2)

    loss = hrow[0, 0]
    class_loss = hrow[0, 1]
    d_vq_loss = commit2[0, 0, 0]
    t_vq_loss = commit2[1, 0, 0]
    d_perp = perp2[0, 0, 0]
    t_perp = perp2[1, 0, 0]
    return (loss, class_loss, d_vq_loss, t_vq_loss, d_perp, t_perp)


# speedup vs baseline: 1.3952x; 1.0075x over previous
import functools
import math

import jax
import jax.numpy as jnp
from jax import lax
from jax.experimental import pallas as pl
from jax.experimental.pallas import tpu as pltpu
from jax.experimental.pallas import tpu_sc as plsc

B = 64; L = 256; D_T = 1280; D_D = 2048; H = 768; OUT = 1024
K_CB = 8192; CODE = 64; NC = 5
KT = 9
LP = L + KT - 1
RSQ = 1.0 / math.sqrt(1.0 + 1e-05)
NVEC = (2 * B * OUT) // CODE
CB_CH = 1024
F32_MIN = float(jnp.finfo(jnp.float32).min)


def _leaky(x):
    return jnp.where(x >= 0, x, 0.01 * x)


def _tenc_body(x_ref, wf_ref, wa_ref, fs_ref, fb_ref, ba_ref, m_ref,
               wsum_ref, gmax_ref):
    xf = x_ref[0].astype(jnp.bfloat16)
    zpad = jnp.zeros((KT // 2, D_T), jnp.bfloat16)
    x = jnp.concatenate([zpad, xf, zpad], axis=0)
    patches = jnp.concatenate([x[k:k + L, :] for k in range(KT)], axis=1)
    accf = jnp.dot(patches, wf_ref[...], preferred_element_type=jnp.float32)
    acca = jnp.dot(patches, wa_ref[...], preferred_element_type=jnp.float32)
    feat = accf * fs_ref[...] + fb_ref[...]
    attn = acca + ba_ref[...]
    m = m_ref[0]
    attn = jnp.where(m > 0, attn, -1e9)
    amax = jnp.max(attn, axis=0, keepdims=True)
    e = jnp.exp(attn - amax)
    w = e / jnp.sum(e, axis=0, keepdims=True)
    wsum_ref[...] = jnp.sum(feat * w, axis=0, keepdims=True)[None]
    fm = jnp.where(m > 0, feat, F32_MIN)
    gmax_ref[...] = jnp.max(fm, axis=0, keepdims=True)[None]


def _tenc(x_pad, wf_r, wa_r, fs, fb, ba2, mask_f, ct):
    nt = H // ct
    return pl.pallas_call(
        _tenc_body,
        grid=(nt, B),
        in_specs=[
            pl.BlockSpec((1, L, D_T), lambda t, b: (b, 0, 0)),
            pl.BlockSpec((KT * D_T, ct), lambda t, b: (0, t)),
            pl.BlockSpec((KT * D_T, ct), lambda t, b: (0, t)),
            pl.BlockSpec((1, ct), lambda t, b: (0, t)),
            pl.BlockSpec((1, ct), lambda t, b: (0, t)),
            pl.BlockSpec((1, ct), lambda t, b: (0, t)),
            pl.BlockSpec((1, L, 1), lambda t, b: (b, 0, 0)),
        ],
        out_specs=[
            pl.BlockSpec((1, 1, ct), lambda t, b: (b, 0, t)),
            pl.BlockSpec((1, 1, ct), lambda t, b: (b, 0, t)),
        ],
        out_shape=[
            jax.ShapeDtypeStruct((B, 1, H), jnp.float32),
            jax.ShapeDtypeStruct((B, 1, H), jnp.float32),
        ],
    )(x_pad, wf_r, wa_r, fs, fb, ba2, mask_f)


def _proj_body(de_ref, wd_ref, co_ref, wp_ref, dsc_ref, dbi_ref, psc_ref,
               pbi_ref, zd_ref, zt_ref):
    a = jnp.dot(de_ref[...], wd_ref[...], preferred_element_type=jnp.float32)
    zd_ref[...] = _leaky(a * dsc_ref[...] + dbi_ref[...])
    b = jnp.dot(co_ref[...], wp_ref[...], preferred_element_type=jnp.float32)
    zt_ref[...] = _leaky(b * psc_ref[...] + pbi_ref[...])


def _proj(de, wd, co, wp, dsc, dbi, psc, pbi):
    return pl.pallas_call(
        _proj_body,
        out_shape=[
            jax.ShapeDtypeStruct((B, OUT), jnp.float32),
            jax.ShapeDtypeStruct((B, OUT), jnp.float32),
        ],
    )(de, wd, co, wp, dsc, dbi, psc, pbi)


def _vq_body(zr_ref, cb_ref, idx_ref, commit_ref, perp_ref):
    n = zr_ref.shape[1]
    zr = zr_ref[0]
    zrb = zr.astype(jnp.bfloat16)
    run_m = jnp.full((n, 1), jnp.inf, jnp.float32)
    run_i = jnp.zeros((n, 1), jnp.int32)
    ones_row = jnp.ones((1, CODE), jnp.float32)
    for c in range(K_CB // CB_CH):
        cbc = cb_ref[0, c * CB_CH:(c + 1) * CB_CH, :]
        d = lax.dot_general(zrb, cbc, (((1,), (1,)), ((), ())),
                            preferred_element_type=jnp.float32)
        cbf = cbc.astype(jnp.float32)
        c2 = lax.dot_general(ones_row, cbf * cbf, (((1,), (1,)), ((), ())),
                             preferred_element_type=jnp.float32)
        s = c2 - 2.0 * d
        m = jnp.min(s, axis=1, keepdims=True)
        jg = lax.broadcasted_iota(jnp.int32, (n, CB_CH), 1) + c * CB_CH
        cand = jnp.min(jnp.where(s == m, jg, jnp.int32(2**31 - 1)),
                       axis=1, keepdims=True)
        better = m < run_m
        run_i = jnp.where(better, cand, run_i)
        run_m = jnp.where(better, m, run_m)
    idx_ref[0] = run_i
    z2 = jnp.sum(zr * zr)
    commit_ref[...] = ((z2 + jnp.sum(run_m)) / float(n * CODE)).reshape(1, 1, 1)
    plogp = jnp.zeros((), jnp.float32)
    for c in range(K_CB // CB_CH):
        bins = lax.broadcasted_iota(jnp.int32, (n, CB_CH), 1) + c * CB_CH
        cnt = jnp.sum((run_i == bins).astype(jnp.float32), axis=0,
                      keepdims=True)
        p = cnt / float(n)
        plogp += jnp.sum(p * jnp.log(p + 1e-10))
    perp_ref[...] = jnp.exp(-plogp).reshape(1, 1, 1)


def _vq(zr2, cb2):
    npath = zr2.shape[0]
    n = zr2.shape[1]
    return pl.pallas_call(
        _vq_body,
        grid=(npath,),
        in_specs=[
            pl.BlockSpec((1, n, CODE), lambda p: (p, 0, 0)),
            pl.BlockSpec((1, K_CB, CODE), lambda p: (p, 0, 0)),
        ],
        out_specs=[
            pl.BlockSpec((1, n, 1), lambda p: (p, 0, 0)),
            pl.BlockSpec((1, 1, 1), lambda p: (p, 0, 0)),
            pl.BlockSpec((1, 1, 1), lambda p: (p, 0, 0)),
        ],
        out_shape=[
            jax.ShapeDtypeStruct((npath, n, 1), jnp.int32),
            jax.ShapeDtypeStruct((npath, 1, 1), jnp.float32),
            jax.ShapeDtypeStruct((npath, 1, 1), jnp.float32),
        ],
    )(zr2, cb2)


_SC_ROWS = NVEC // 32


def _sc_gather_body(idx_hbm, cbd_hbm, cbt_hbm, q_hbm, idx_v, rows_v, sem):
    wid = lax.axis_index("s") * 2 + lax.axis_index("c")
    base = wid * _SC_ROWS
    pltpu.sync_copy(idx_hbm.at[pl.ds(base, _SC_ROWS)], idx_v)

    @pl.when(wid < 16)
    def _():
        pltpu.async_copy(cbd_hbm.at[idx_v], rows_v, sem).wait()

    @pl.when(wid >= 16)
    def _():
        pltpu.async_copy(cbt_hbm.at[idx_v], rows_v, sem).wait()

    pltpu.sync_copy(rows_v, q_hbm.at[pl.ds(base, _SC_ROWS)])


@functools.lru_cache(maxsize=1)
def _sc_gather_built():
    return pl.kernel(
        _sc_gather_body,
        mesh=plsc.VectorSubcoreMesh(core_axis_name="c", subcore_axis_name="s"),
        out_type=jax.ShapeDtypeStruct((NVEC, 128), jnp.float32),
        scratch_types=[
            pltpu.VMEM((_SC_ROWS,), jnp.int32),
            pltpu.VMEM((_SC_ROWS, 128), jnp.float32),
            pltpu.SemaphoreType.DMA,
        ],
    )


def _sc_gather(idx_all, cbd, cbt):
    return _sc_gather_built()(idx_all, cbd, cbt)


def _digamma1(x):
    acc = jnp.zeros_like(x)
    for k in range(8):
        acc += 1.0 / (x + float(k))
    y = x + 8.0
    inv = 1.0 / y
    inv2 = inv * inv
    tail = inv2 * (1.0 / 12.0 - inv2 * (1.0 / 120.0 - inv2 * (1.0 / 252.0)))
    return jnp.log(y) - 0.5 * inv - tail - acc


def _gammaln1(x):
    acc = jnp.zeros_like(x)
    for k in range(8):
        acc += jnp.log(x + float(k))
    y = x + 8.0
    inv = 1.0 / y
    inv2 = inv * inv
    tail = inv * (1.0 / 12.0 - inv2 * (1.0 / 360.0 - inv2 * (1.0 / 1260.0)))
    return (y - 0.5) * jnp.log(y) - y + 0.9189385332046727 + tail - acc


_LGAMMA_NC = math.lgamma(float(NC))


def _head_body(fu_ref, we1_ref, esc_ref, ebi_ref, we2_ref, be2_ref, y_ref,
               cm_ref, out_ref):
    h = jnp.dot(fu_ref[...], we1_ref[...], preferred_element_type=jnp.float32)
    h = _leaky(h * esc_ref[...] + ebi_ref[...])
    logits = jnp.dot(h, we2_ref[...], preferred_element_type=jnp.float32)
    logits = logits + be2_ref[...]
    col = lax.broadcasted_iota(jnp.int32, (B, 128), 1)
    valid = col < NC
    sp = jnp.maximum(logits, 0.0) + jnp.log1p(jnp.exp(-jnp.abs(logits)))
    alpha = sp + 1.0
    alpha_s = jnp.where(valid, alpha, 1.0)
    y_oh = (col == y_ref[...]).astype(jnp.float32)
    S = jnp.sum(jnp.where(valid, alpha, 0.0), axis=1, keepdims=True)
    err = jnp.sum(y_oh * (_digamma1(S) - _digamma1(alpha_s)), axis=1,
                  keepdims=True)
    a_t = jnp.where(valid, y_oh + (1.0 - y_oh) * alpha_s, 1.0)
    St = jnp.sum(jnp.where(valid, a_t, 0.0), axis=1, keepdims=True)
    kl = (_gammaln1(St)
          - jnp.sum(jnp.where(valid, _gammaln1(a_t), 0.0), axis=1,
                    keepdims=True)
          - _LGAMMA_NC
          + jnp.sum(jnp.where(valid, (a_t - 1.0)
                              * (_digamma1(a_t) - _digamma1(St)), 0.0),
                    axis=1, keepdims=True))
    class_loss = jnp.sum(err + 0.1 * kl) / float(B)
    loss = class_loss + cm_ref[0, 0, 0] + cm_ref[1, 0, 0]
    out_ref[...] = jnp.where(col[0:1, :] == 0, loss, class_loss)


def _head(fused, we1, esc, ebi, we2p, be2p, y2, commit):
    return pl.pallas_call(
        _head_body,
        in_specs=[
            pl.BlockSpec((B, 2 * OUT), lambda: (0, 0)),
            pl.BlockSpec((2 * OUT, 512), lambda: (0, 0)),
            pl.BlockSpec((1, 512), lambda: (0, 0)),
            pl.BlockSpec((1, 512), lambda: (0, 0)),
            pl.BlockSpec((512, 128), lambda: (0, 0)),
            pl.BlockSpec((1, 128), lambda: (0, 0)),
            pl.BlockSpec((B, 1), lambda: (0, 0)),
            pl.BlockSpec(memory_space=pltpu.SMEM),
        ],
        out_shape=jax.ShapeDtypeStruct((1, 128), jnp.float32),
    )(fused, we1, esc, ebi, we2p, be2p, y2, commit)


def kernel(d_emb, t_emb, t_mask, y, W_d, b_d, dg, db, Wf, bf, fg, fb, Wa, ba,
           Wp, bp, pg, pb, cb_d, cb_t, We1, be1, eg, eb, We2, be2):
    f32 = jnp.float32
    bf16 = jnp.bfloat16

    wf_r = jnp.transpose(Wf.astype(bf16), (2, 1, 0)).reshape(KT * D_T, H)
    wa_r = jnp.transpose(Wa.astype(bf16), (2, 1, 0)).reshape(KT * D_T, H)
    fs = (fg * RSQ).reshape(1, H)
    fbv = (bf * fg * RSQ + fb).reshape(1, H)
    ba2 = ba.reshape(1, H)
    mask_f = t_mask.astype(f32).reshape(B, L, 1)
    wsum, gmax = _tenc(t_emb, wf_r, wa_r, fs, fbv, ba2, mask_f, ct=H)
    comb = jnp.concatenate([wsum.reshape(B, H), gmax.reshape(B, H)], axis=-1)

    dsc = (dg * RSQ).reshape(1, OUT)
    dbi = (b_d * dg * RSQ + db).reshape(1, OUT)
    psc = (pg * RSQ).reshape(1, OUT)
    pbi = (bp * pg * RSQ + pb).reshape(1, OUT)
    z_d, z_t = _proj(d_emb, W_d, comb, Wp, dsc, dbi, psc, pbi)

    zr2 = jnp.stack([z_d.reshape(-1, CODE), z_t.reshape(-1, CODE)])
    cb2 = jnp.stack([cb_d, cb_t]).astype(bf16)
    idx2, commit2, perp2 = _vq(zr2, cb2)

    idx_all = idx2.reshape(NVEC)
    cbd_p = jnp.pad(cb_d, ((0, 0), (0, 128 - CODE)))
    cbt_p = jnp.pad(cb_t, ((0, 0), (0, 128 - CODE)))
    q_all = _sc_gather(idx_all, cbd_p, cbt_p)[:, :CODE]
    q_d = q_all[:NVEC // 2].reshape(B, OUT)
    q_t = q_all[NVEC // 2:].reshape(B, OUT)

    fused = jnp.concatenate([q_d, q_t], axis=-1)
    esc = (eg * RSQ).reshape(1, 512)
    ebi = (be1 * eg * RSQ + eb).reshape(1, 512)
    we2p = jnp.pad(We2, ((0, 0), (0, 128 - NC)))
    be2p = jnp.pad(be2, (0, 128 - NC)).reshape(1, 128)
    y2 = y.astype(jnp.int32).reshape(B, 1)
    hrow = _head(fused, We1, esc, ebi, we2p, be2p, y2, commit2)

    loss = hrow[0, 0]
    class_loss = hrow[0, 1]
    d_vq_loss = commit2[0, 0, 0]
    t_vq_loss = commit2[1, 0, 0]
    d_perp = perp2[0, 0, 0]
    t_perp = perp2[1, 0, 0]
    return (loss, class_loss, d_vq_loss, t_vq_loss, d_perp, t_perp)
